# Initial kernel scaffold; baseline (speedup 1.0000x reference)
import jax
import jax.numpy as jnp
from jax.experimental import pallas as pl


def kernel(xyz, params):
    raise NotImplementedError("write your pallas kernel here")


# speedup vs baseline: 6.1434x; 6.1434x over previous
import functools

import jax
import jax.numpy as jnp
from jax.experimental import pallas as pl
from jax.experimental.pallas import tpu as pltpu

F32 = jnp.float32
EPS = 1e-5
HI = jax.lax.Precision.HIGHEST


def _dot(a, b):
    return jax.lax.dot_general(a, b, (((1,), (0,)), ((), ())),
                               preferred_element_type=F32, precision=HI)


def _dot_def(a, b):
    return jax.lax.dot_general(a.astype(jnp.bfloat16), b.astype(jnp.bfloat16),
                               (((1,), (0,)), ((), ())),
                               preferred_element_type=F32)


def _bf(x):
    return x.astype(jnp.bfloat16).astype(F32)


def _colsum_tree(x):
    while x.shape[0] > 1:
        h = x.shape[0] // 2
        x = x[:h, :] + x[h:, :]
    return x


def _colsum_mxu(x):
    ones = jnp.ones((8, x.shape[0]), F32)
    return _dot(ones, x)[0:1, :]


def _bn_terms(sp_ref, cp_ref, cnt):
    mean = _colsum_mxu(sp_ref[:, 0, :]) * (1.0 / cnt)
    var = _colsum_mxu(cp_ref[:, 0, :]) * (1.0 / cnt)
    return mean, jnp.sqrt(var + EPS)


def _bn_apply(x, mean, sd, gamma, beta):
    return ((x - mean) / sd) * gamma + beta


def _stats2_body(x_ref, sp_ref, cp_ref, *, cnt):
    mean = _colsum_mxu(sp_ref[:, 0, :]) * (1.0 / cnt)
    xc = x_ref[0] - mean
    cp_ref[0] = _colsum_mxu(xc * xc)


def _center_stats(x, sp, cnt, bm=1024):
    b, n, _ = x.shape
    bm = min(bm, n)
    rr = n // bm
    sn = sp.shape[0]
    return pl.pallas_call(
        functools.partial(_stats2_body, cnt=cnt),
        grid=(b, rr),
        in_specs=[pl.BlockSpec((1, bm, 128), lambda bi, r: (bi, r, 0)),
                  pl.BlockSpec((sn, 1, 128), lambda bi, r: (0, 0, 0))],
        out_specs=pl.BlockSpec((1, 1, 128), lambda bi, r, rr=rr: (bi * rr + r, 0, 0)),
        out_shape=jax.ShapeDtypeStruct((b * rr, 1, 128), F32),
    )(x, sp)


def _knn_body(q_ref, db_ref, idx_ref, *, n_db, bm):
    q = q_ref[0]
    db = db_ref[0]
    x2 = jnp.sum(db * db, axis=0, keepdims=True)
    inner = jax.lax.dot_general(q.astype(jnp.bfloat16), db.astype(jnp.bfloat16),
                                (((1,), (0,)), ((), ())),
                                preferred_element_type=F32)
    d = x2 - 2.0 * inner
    lane = jax.lax.broadcasted_iota(jnp.int32, (bm, n_db), 1)
    cols = []
    for _ in range(8):
        m = jnp.min(d, axis=1, keepdims=True)
        sel = jnp.min(jnp.where(d <= m, lane, n_db), axis=1, keepdims=True)
        cols.append(sel)
        d = jnp.where(lane == sel, jnp.float32(3.0e38), d)
    idx_ref[0] = jnp.concatenate(cols, axis=1)


def _knn(qpad, db_pad, bm=256):
    b, m, _ = qpad.shape
    n_db = db_pad.shape[2]
    return pl.pallas_call(
        functools.partial(_knn_body, n_db=n_db, bm=bm),
        grid=(b, m // bm),
        in_specs=[pl.BlockSpec((1, bm, 128), lambda bi, r: (bi, r, 0)),
                  pl.BlockSpec((1, 128, n_db), lambda bi, r: (bi, 0, 0))],
        out_specs=pl.BlockSpec((1, bm, 8), lambda bi, r: (bi, r, 0)),
        out_shape=jax.ShapeDtypeStruct((b, m, 8), jnp.int32),
    )(qpad, db_pad)


def _p1_body(xyzT_ref, idx_ref, w1t_ref, y1_ref, sp_ref, *, n, bm):
    r = pl.program_id(1)
    xyzT = xyzT_ref[0]
    xc = xyzT_ref[0, pl.ds(r * bm, bm), :]
    idxb = idx_ref[0]
    lane = jax.lax.broadcasted_iota(jnp.int32, (bm, n), 1)
    parts = []
    for k in range(8):
        oh = (lane == idxb[:, k:k + 1]).astype(F32)
        gk = _dot(oh, xyzT) - xc
        yk = _dot_def(gk, w1t_ref[...])
        y1_ref[0, k] = yk
        parts.append(_colsum_mxu(yk))
    sp_ref[0] = _colsum_mxu(jnp.concatenate(parts, axis=0))


def _p1(xyzT_pad, idx, w1t_pad, bm=256):
    b, n, _ = xyzT_pad.shape
    rr = n // bm
    return pl.pallas_call(
        functools.partial(_p1_body, n=n, bm=bm),
        grid=(b, rr),
        in_specs=[pl.BlockSpec((1, n, 128), lambda bi, r: (bi, 0, 0)),
                  pl.BlockSpec((1, bm, 8), lambda bi, r: (bi, r, 0)),
                  pl.BlockSpec((128, 128), lambda bi, r: (0, 0))],
        out_specs=[pl.BlockSpec((1, 8, bm, 128), lambda bi, r: (bi, 0, r, 0)),
                   pl.BlockSpec((1, 1, 128), lambda bi, r, rr=rr: (bi * rr + r, 0, 0))],
        out_shape=[jax.ShapeDtypeStruct((b, 8, n, 128), F32),
                   jax.ShapeDtypeStruct((b * rr, 1, 128), F32)],
    )(xyzT_pad, idx, w1t_pad)


def _p23_body(y_ref, sp_ref, cp_ref, wt_ref, gb_ref, out_ref, spo_ref, *, cnt):
    mean, sd = _bn_terms(sp_ref, cp_ref, cnt)
    h = jnp.maximum(_bn_apply(y_ref[0, 0], mean, sd, gb_ref[0:1, :], gb_ref[1:2, :]), 0.0)
    out = _dot_def(h, wt_ref[...])
    out_ref[0, 0] = out
    spo_ref[0] = _colsum_mxu(out)


def _p23(y, sp, cp, wt, gb, cnt, bmr):
    b, kk, n, _ = y.shape
    bmr = min(bmr, n)
    rr = n // bmr
    sn, cn = sp.shape[0], cp.shape[0]
    return pl.pallas_call(
        functools.partial(_p23_body, cnt=cnt),
        grid=(b, kk, rr),
        in_specs=[pl.BlockSpec((1, 1, bmr, 128), lambda bi, ki, r: (bi, ki, r, 0)),
                  pl.BlockSpec((sn, 1, 128), lambda bi, ki, r: (0, 0, 0)),
                  pl.BlockSpec((cn, 1, 128), lambda bi, ki, r: (0, 0, 0)),
                  pl.BlockSpec((128, 128), lambda bi, ki, r: (0, 0)),
                  pl.BlockSpec((8, 128), lambda bi, ki, r: (0, 0))],
        out_specs=[pl.BlockSpec((1, 1, bmr, 128), lambda bi, ki, r: (bi, ki, r, 0)),
                   pl.BlockSpec((1, 1, 128),
                                lambda bi, ki, r, kk=kk, rr=rr: ((bi * kk + ki) * rr + r, 0, 0))],
        out_shape=[jax.ShapeDtypeStruct((b, kk, n, 128), F32),
                   jax.ShapeDtypeStruct((b * kk * rr, 1, 128), F32)],
    )(y, sp, cp, wt, gb)


def _p4_body(y_ref, sp_ref, cp_ref, gb_ref, out_ref, spo_ref, *, cnt):
    mean, sd = _bn_terms(sp_ref, cp_ref, cnt)
    g, be = gb_ref[0:1, :], gb_ref[1:2, :]
    p = jnp.maximum(_bn_apply(y_ref[0, 0], mean, sd, g, be), 0.0)
    for k in range(1, 8):
        p = jnp.maximum(p, jnp.maximum(_bn_apply(y_ref[0, k], mean, sd, g, be), 0.0))
    out_ref[0] = p
    spo_ref[0] = _colsum_mxu(p)


def _p4(y, sp, cp, gb, cnt, bm=512):
    b, _, n, _ = y.shape
    bm = min(bm, n)
    rr = n // bm
    sn, cn = sp.shape[0], cp.shape[0]
    return pl.pallas_call(
        functools.partial(_p4_body, cnt=cnt),
        grid=(b, rr),
        in_specs=[pl.BlockSpec((1, 8, bm, 128), lambda bi, r: (bi, 0, r, 0)),
                  pl.BlockSpec((sn, 1, 128), lambda bi, r: (0, 0, 0)),
                  pl.BlockSpec((cn, 1, 128), lambda bi, r: (0, 0, 0)),
                  pl.BlockSpec((8, 128), lambda bi, r: (0, 0))],
        out_specs=[pl.BlockSpec((1, bm, 128), lambda bi, r: (bi, r, 0)),
                   pl.BlockSpec((1, 1, 128), lambda bi, r, rr=rr: (bi * rr + r, 0, 0))],
        out_shape=[jax.ShapeDtypeStruct((b, n, 128), F32),
                   jax.ShapeDtypeStruct((b * rr, 1, 128), F32)],
    )(y, sp, cp, gb)


def _onehot_sum(idxb, bm, n):
    lane = jax.lax.broadcasted_iota(jnp.int32, (bm, n), 1)
    oh = (lane == idxb[:, 0:1]).astype(F32)
    for k in range(1, 8):
        oh = oh + (lane == idxb[:, k:k + 1]).astype(F32)
    return oh


def _kblock_body(pts_ref, idx_ref, sp_ref, cp_ref, wat_ref, wbt_ref, bnp_ref,
                 xyzrep_ref, uct_ref, unt_ref,
                 out_ref, spo_ref, new6_ref, ptsn_sc,
                 *, n, bm, cnt, last, points_out):
    r = pl.program_id(1)
    mean, sd = _bn_terms(sp_ref, cp_ref, cnt)

    @pl.when(r == 0)
    def _():
        ptsn_sc[...] = _bf(jnp.maximum(
            _bn_apply(pts_ref[0], mean, sd, bnp_ref[0:1, :], bnp_ref[1:2, :]), 0.0))

    oh = _onehot_sum(idx_ref[0], bm, n)
    s_rows = _dot(oh, ptsn_sc[...])
    p_rows = ptsn_sc[pl.ds(r * bm, bm), :]
    if points_out:
        shortcut = pts_ref[0, pl.ds(r * bm, bm), :]
        out = (_dot_def(p_rows, wat_ref[...]) + _dot(s_rows, _bf(wbt_ref[...]))) \
            * jnp.float32(1.0 / 9.0) + bnp_ref[2:3, :] + shortcut
        out_ref[0] = out
        spo_ref[0] = _colsum_mxu(out)
    if last:
        o6 = (_dot_def(p_rows, uct_ref[...]) + _dot(s_rows, _bf(unt_ref[...]))) \
            * jnp.float32(1.0 / 9.0) + bnp_ref[3:4, 0:8] + xyzrep_ref[0]
        new6_ref[0] = o6


def _kblock(pts, idx, sp, cp, wat, wbt, bnp, xyzrep, uct, unt, cnt,
            last, points_out, bm=256):
    b, n, _ = pts.shape
    rr = n // bm
    sn, cn = sp.shape[0], cp.shape[0]
    out_specs = []
    out_shape = []
    if points_out:
        out_specs += [pl.BlockSpec((1, bm, 128), lambda bi, r: (bi, r, 0)),
                      pl.BlockSpec((1, 1, 128), lambda bi, r, rr=rr: (bi * rr + r, 0, 0))]
        out_shape += [jax.ShapeDtypeStruct((b, n, 128), F32),
                      jax.ShapeDtypeStruct((b * rr, 1, 128), F32)]
    if last:
        out_specs += [pl.BlockSpec((1, bm, 8), lambda bi, r: (bi, r, 0))]
        out_shape += [jax.ShapeDtypeStruct((b, n, 8), F32)]

    def body(pts_ref, idx_ref, sp_ref, cp_ref, wat_ref, wbt_ref, bnp_ref,
             xyzrep_ref, uct_ref, unt_ref, *rest):
        outs = list(rest[:-1])
        sc = rest[-1]
        o = outs.pop(0) if points_out else None
        s = outs.pop(0) if points_out else None
        n6 = outs.pop(0) if last else None
        _kblock_body(pts_ref, idx_ref, sp_ref, cp_ref, wat_ref, wbt_ref,
                     bnp_ref, xyzrep_ref, uct_ref, unt_ref, o, s, n6, sc,
                     n=n, bm=bm, cnt=cnt, last=last, points_out=points_out)

    res = pl.pallas_call(
        body,
        grid=(b, rr),
        in_specs=[pl.BlockSpec((1, n, 128), lambda bi, r: (bi, 0, 0)),
                  pl.BlockSpec((1, bm, 8), lambda bi, r: (bi, r, 0)),
                  pl.BlockSpec((sn, 1, 128), lambda bi, r: (0, 0, 0)),
                  pl.BlockSpec((cn, 1, 128), lambda bi, r: (0, 0, 0)),
                  pl.BlockSpec((128, 128), lambda bi, r: (0, 0)),
                  pl.BlockSpec((128, 128), lambda bi, r: (0, 0)),
                  pl.BlockSpec((8, 128), lambda bi, r: (0, 0)),
                  pl.BlockSpec((1, bm, 8), lambda bi, r: (bi, r, 0)),
                  pl.BlockSpec((128, 8), lambda bi, r: (0, 0)),
                  pl.BlockSpec((128, 8), lambda bi, r: (0, 0))],
        out_specs=out_specs,
        out_shape=out_shape,
        scratch_shapes=[pltpu.VMEM((n, 128), F32)],
    )(pts, idx, sp, cp, wat, wbt, bnp, xyzrep, uct, unt)
    return res


def _gm_body(pts_ref, idx_ref, out_ref, spo_ref, *, n_src, bm):
    oh = _onehot_sum(idx_ref[0], bm, n_src)
    out = _dot(oh, pts_ref[0]) * jnp.float32(1.0 / 8.0)
    out_ref[0] = out
    spo_ref[0] = _colsum_mxu(out)


def _gather_mean(pts, idx, bm=256):
    b, n_src, _ = pts.shape
    m = idx.shape[1]
    rr = m // bm
    return pl.pallas_call(
        functools.partial(_gm_body, n_src=n_src, bm=bm),
        grid=(b, rr),
        in_specs=[pl.BlockSpec((1, n_src, 128), lambda bi, r: (bi, 0, 0)),
                  pl.BlockSpec((1, bm, 8), lambda bi, r: (bi, r, 0))],
        out_specs=[pl.BlockSpec((1, bm, 128), lambda bi, r: (bi, r, 0)),
                   pl.BlockSpec((1, 1, 128), lambda bi, r, rr=rr: (bi * rr + r, 0, 0))],
        out_shape=[jax.ShapeDtypeStruct((b, m, 128), F32),
                   jax.ShapeDtypeStruct((b * rr, 1, 128), F32)],
    )(pts, idx)


def _pack_rows(*rows):
    out = jnp.zeros((8, 128), F32)
    for i, row in enumerate(rows):
        out = out.at[i, :row.shape[0]].set(row)
    return out


def _ru_stage(pts, sp, cp, idx, rp, xyzrep, cnt, final):
    uct = jnp.pad(rp['uc_w'].T, ((0, 0), (0, 2)))
    unt = jnp.pad(rp['un_w'].T, ((0, 0), (0, 2)))
    c6 = (rp['uc_b'] + 8.0 * rp['un_b']) * (1.0 / 9.0)
    new6 = None
    for i in range(12):
        blk = rp['blocks'][i]
        c0 = (blk['ba'] + 8.0 * blk['bb']) * (1.0 / 9.0)
        bnp = _pack_rows(blk['bn_g'], blk['bn_b'], c0, c6)
        last = (i == 11)
        points_out = not (last and final)
        res = _kblock(pts, idx, sp, cp, blk['wa'].T, blk['wb'].T, bnp, xyzrep,
                      uct, unt, cnt, last, points_out)
        if points_out and last:
            pts, sp, new6 = res
        elif points_out:
            pts, sp = res
        else:
            new6 = res[0]
        if points_out and not last:
            cp = _center_stats(pts, sp, cnt)
    return pts, new6


def _assemble(new6x, b, n):
    t = new6x[:, :, :6].reshape(b, n, 3, 2)
    t = jnp.transpose(t, (0, 2, 3, 1))
    return t.reshape(b, 3, 2 * n)


def _coord_layouts(xyz3n):
    b, _, n = xyz3n.shape
    xyzT = jnp.transpose(xyz3n, (0, 2, 1))
    qpad = jnp.pad(xyzT, ((0, 0), (0, 0), (0, 125)))
    dbpad = jnp.pad(xyz3n, ((0, 0), (0, 125), (0, 0)))
    xyzrep = jnp.pad(jnp.repeat(xyzT, 2, axis=2), ((0, 0), (0, 0), (0, 2)))
    return qpad, dbpad, xyzrep


def kernel(xyz, params):
    b, _, n = xyz.shape
    qpad0, db0, xyzrep0 = _coord_layouts(xyz)

    idx1 = _knn(qpad0, db0)

    feat = params['feat']
    w1t = jnp.pad(feat['w1'].T, ((0, 125), (0, 0)))
    y1, sp = _p1(qpad0, idx1, w1t)
    cnt_f = float(b * 8 * n)
    cp = _center_stats(y1.reshape(b, 8 * n, 128), sp, cnt_f)
    y2, sp = _p23(y1, sp, cp, feat['w2'].T, _pack_rows(feat['g1'], feat['bb1']),
                  cnt_f, bmr=2048)
    cp = _center_stats(y2.reshape(b, 8 * n, 128), sp, cnt_f)
    y3, sp = _p23(y2, sp, cp, feat['w3'].T, _pack_rows(feat['g2'], feat['bb2']),
                  cnt_f, bmr=2048)
    cp = _center_stats(y3.reshape(b, 8 * n, 128), sp, cnt_f)
    pts, sp = _p4(y3, sp, cp, _pack_rows(feat['g3'], feat['bb3']), cnt_f)
    cp = _center_stats(pts, sp, float(b * n))

    pts, new6 = _ru_stage(pts, sp, cp, idx1, params['ru1'], xyzrep0,
                          cnt=float(b * n), final=False)
    nx1 = _assemble(new6, b, n)
    qpad1, db1, xyzrep1 = _coord_layouts(nx1)

    idx2 = _knn(qpad1, db0)
    pts, sp = _gather_mean(pts, idx2)
    cp = _center_stats(pts, sp, float(b * 2 * n))
    idx3 = _knn(qpad1, db1)
    _, new6b = _ru_stage(pts, sp, cp, idx3, params['ru2'], xyzrep1,
                         cnt=float(b * 2 * n), final=True)
    return _assemble(new6b, b, 2 * n)
